# Initial kernel scaffold; baseline (speedup 1.0000x reference)
import jax
import jax.numpy as jnp
from jax.experimental import pallas as pl


def kernel(node_feat, bond_feat, connectivity, batchAssign, W0, b0, W1, b1, W2, b2, Wh0, bh0, Wh1, bh1, Wi, bi, We, be, Wlb, blb, Wub, bub):
    raise NotImplementedError("write your pallas kernel here")


# speedup vs baseline: 693.2948x; 693.2948x over previous
import functools

import jax
import jax.numpy as jnp
from jax.experimental import pallas as pl
from jax.experimental.pallas import tpu as pltpu

_N = 512
_D = 640


def _model_kernel(node_ref, bond_ref, conn_ref,
                  w0_ref, b0_ref, w1_ref, b1_ref, w2_ref, b2_ref,
                  wh0_ref, bh0_ref, wh1_ref, bh1_ref, wcat_ref, bcat_ref,
                  out_ref):
    f32 = jnp.float32
    dot = functools.partial(
        jax.lax.dot_general,
        preferred_element_type=f32,
        precision=jax.lax.Precision.HIGHEST,
    )

    a = (conn_ref[0] != 0).astype(f32)
    row = jax.lax.broadcasted_iota(jnp.int32, (_N, _N), 0)
    col = jax.lax.broadcasted_iota(jnp.int32, (_N, _N), 1)
    ahat = a + (row == col).astype(f32)
    deg = 1.0 + jnp.sum(a, axis=0, keepdims=True)
    dinv = jax.lax.rsqrt(deg)
    di_col = dinv.reshape(_N, 1)

    x = jnp.concatenate([node_ref[0], bond_ref[0]], axis=1)

    for w_ref, b_ref in ((w0_ref, b0_ref), (w1_ref, b1_ref), (w2_ref, b2_ref)):
        h = dot(x, w_ref[...], (((1,), (1,)), ((), ())))
        g = di_col * h
        t = dot(ahat, g, (((0,), (0,)), ((), ())))
        x = di_col * t + b_ref[...]

    pooled = jnp.sum(x, axis=0, keepdims=True)
    h = jax.nn.softplus(pooled)
    h = dot(h, wh0_ref[...], (((1,), (1,)), ((), ()))) + bh0_ref[...]
    h = dot(h, wh1_ref[...], (((1,), (1,)), ((), ()))) + bh1_ref[...]
    res = dot(h, wcat_ref[...], (((1,), (1,)), ((), ()))) + bcat_ref[...]
    out_ref[...] = res.reshape(1, 1, 128)


def kernel(node_feat, bond_feat, connectivity, batchAssign,
           W0, b0, W1, b1, W2, b2, Wh0, bh0, Wh1, bh1,
           Wi, bi, We, be, Wlb, blb, Wub, bub):
    S, N, P = node_feat.shape
    d = P + N

    Wcat = jnp.concatenate([Wi, We, Wlb, Wub], axis=0)
    pad = 128 - Wcat.shape[0]
    Wcat = jnp.pad(Wcat, ((0, pad), (0, 0)))
    bcat = jnp.pad(jnp.concatenate([bi, be, blb, bub]), (0, pad)).reshape(1, 128)

    b0r = b0.reshape(1, d)
    b1r = b1.reshape(1, d)
    b2r = b2.reshape(1, d)
    bh0r = bh0.reshape(1, 256)
    bh1r = bh1.reshape(1, 128)

    full = lambda shape: pl.BlockSpec(shape, lambda s: (0,) * len(shape))
    grid_spec = pl.GridSpec(
        grid=(S,),
        in_specs=[
            pl.BlockSpec((1, N, P), lambda s: (s, 0, 0)),
            pl.BlockSpec((1, N, N), lambda s: (s, 0, 0)),
            pl.BlockSpec((1, N, N), lambda s: (s, 0, 0)),
            full((d, d)), full((1, d)),
            full((d, d)), full((1, d)),
            full((d, d)), full((1, d)),
            full((256, d)), full((1, 256)),
            full((128, 256)), full((1, 128)),
            full((128, 128)), full((1, 128)),
        ],
        out_specs=pl.BlockSpec((1, 1, 128), lambda s: (s, 0, 0)),
    )

    out = pl.pallas_call(
        _model_kernel,
        grid_spec=grid_spec,
        out_shape=jax.ShapeDtypeStruct((S, 1, 128), jnp.float32),
        compiler_params=pltpu.CompilerParams(
            dimension_semantics=("arbitrary",),
        ),
    )(node_feat, bond_feat, connectivity,
      W0, b0r, W1, b1r, W2, b2r, Wh0, bh0r, Wh1, bh1r, Wcat, bcat)

    out = out.reshape(S, 128)
    return (out[:, :100], out[:, 100:101], out[:, 101:102], out[:, 102:103])


# speedup vs baseline: 1427.7358x; 2.0593x over previous
import functools

import jax
import jax.numpy as jnp
from jax.experimental import pallas as pl
from jax.experimental.pallas import tpu as pltpu

_N = 512
_D = 640


def _model_kernel(node_ref, bond_ref, conn_ref,
                  w0_ref, b0_ref, w1_ref, b1_ref, w2_ref, b2_ref,
                  wh0_ref, bh0_ref, wh1_ref, bh1_ref, wcat_ref, bcat_ref,
                  out_ref):
    f32 = jnp.float32
    bf16 = jnp.bfloat16
    dot = functools.partial(
        jax.lax.dot_general,
        preferred_element_type=f32,
        precision=jax.lax.Precision.DEFAULT,
    )

    a = (conn_ref[0] != 0).astype(f32)
    row = jax.lax.broadcasted_iota(jnp.int32, (_N, _N), 0)
    col = jax.lax.broadcasted_iota(jnp.int32, (_N, _N), 1)
    ahat = a + (row == col).astype(f32)
    deg = 1.0 + jnp.sum(a, axis=0, keepdims=True)
    dinv = 1.0 / jnp.sqrt(deg)
    di_col = dinv.reshape(_N, 1)

    x = jnp.concatenate([node_ref[0], bond_ref[0]], axis=1)

    for w_ref, b_ref in ((w0_ref, b0_ref), (w1_ref, b1_ref), (w2_ref, b2_ref)):
        h = dot(x, w_ref[...], (((1,), (1,)), ((), ())))
        g = di_col * h
        g1 = g.astype(bf16).astype(f32)
        r1 = g - g1
        g2 = r1.astype(bf16).astype(f32)
        g3 = r1 - g2
        t = (dot(ahat, g1, (((0,), (0,)), ((), ()))) +
             dot(ahat, g2, (((0,), (0,)), ((), ()))) +
             dot(ahat, g3, (((0,), (0,)), ((), ()))))
        x = di_col * t + b_ref[...]

    pooled = jnp.sum(x, axis=0, keepdims=True)
    h = jax.nn.softplus(pooled)
    h = dot(h, wh0_ref[...], (((1,), (1,)), ((), ()))) + bh0_ref[...]
    h = dot(h, wh1_ref[...], (((1,), (1,)), ((), ()))) + bh1_ref[...]
    res = dot(h, wcat_ref[...], (((1,), (1,)), ((), ()))) + bcat_ref[...]
    out_ref[...] = res.reshape(1, 1, 128)


def kernel(node_feat, bond_feat, connectivity, batchAssign,
           W0, b0, W1, b1, W2, b2, Wh0, bh0, Wh1, bh1,
           Wi, bi, We, be, Wlb, blb, Wub, bub):
    S, N, P = node_feat.shape
    d = P + N

    Wcat = jnp.concatenate([Wi, We, Wlb, Wub], axis=0)
    pad = 128 - Wcat.shape[0]
    Wcat = jnp.pad(Wcat, ((0, pad), (0, 0)))
    bcat = jnp.pad(jnp.concatenate([bi, be, blb, bub]), (0, pad)).reshape(1, 128)

    b0r = b0.reshape(1, d)
    b1r = b1.reshape(1, d)
    b2r = b2.reshape(1, d)
    bh0r = bh0.reshape(1, 256)
    bh1r = bh1.reshape(1, 128)

    full = lambda shape: pl.BlockSpec(shape, lambda s: (0,) * len(shape))
    grid_spec = pl.GridSpec(
        grid=(S,),
        in_specs=[
            pl.BlockSpec((1, N, P), lambda s: (s, 0, 0)),
            pl.BlockSpec((1, N, N), lambda s: (s, 0, 0)),
            pl.BlockSpec((1, N, N), lambda s: (s, 0, 0)),
            full((d, d)), full((1, d)),
            full((d, d)), full((1, d)),
            full((d, d)), full((1, d)),
            full((256, d)), full((1, 256)),
            full((128, 256)), full((1, 128)),
            full((128, 128)), full((1, 128)),
        ],
        out_specs=pl.BlockSpec((1, 1, 128), lambda s: (s, 0, 0)),
    )

    out = pl.pallas_call(
        _model_kernel,
        grid_spec=grid_spec,
        out_shape=jax.ShapeDtypeStruct((S, 1, 128), jnp.float32),
        compiler_params=pltpu.CompilerParams(
            dimension_semantics=("arbitrary",),
        ),
    )(node_feat, bond_feat, connectivity,
      W0, b0r, W1, b1r, W2, b2r, Wh0, bh0r, Wh1, bh1r, Wcat, bcat)

    out = out.reshape(S, 128)
    return (out[:, :100], out[:, 100:101], out[:, 101:102], out[:, 102:103])
